# baseline (speedup 1.0000x reference)
import functools

import jax
import jax.numpy as jnp
import numpy as np
from jax import lax
from jax.experimental import pallas as pl
from jax.experimental.pallas import tpu as pltpu
from jax.experimental.pallas import tpu_sc as plsc

N = 4096
D = 128
LANES = 16

NC = 2
NS = 16
NW = NC * NS
B_PER_W = N // NW

SPLIT = 3072
ROW_BLK = 512
TC_GRID = SPLIT // ROW_BLK
NSTREAMS = 2
CW = N // NSTREAMS

SC_ROWS = N - SPLIT
RPT = SC_ROWS // NW
RCHUNK = 8


def _tc_degree_body(*refs):
    adj_refs = refs[:NSTREAMS]
    outd_ref, ind_ref = refs[NSTREAMS], refs[NSTREAMS + 1]
    i = pl.program_id(0)
    blks = [r[...] for r in adj_refs]
    row = jnp.sum(blks[0], axis=1)
    for b in blks[1:]:
        row = row + jnp.sum(b, axis=1)
    outd_ref[...] = row.astype(jnp.int32)
    cols = [jnp.sum(b, axis=0) for b in blks]

    @pl.when(i == 0)
    def _():
        for s in range(NSTREAMS):
            ind_ref[pl.ds(s * CW, CW)] = cols[s]

    @pl.when(i > 0)
    def _():
        for s in range(NSTREAMS):
            ind_ref[pl.ds(s * CW, CW)] = ind_ref[pl.ds(s * CW, CW)] + cols[s]


def _tc_degrees(adj):
    def make_spec(s):
        return pl.BlockSpec((ROW_BLK, CW), lambda i, s=s: (i, s))

    return pl.pallas_call(
        _tc_degree_body,
        grid=(TC_GRID,),
        in_specs=[make_spec(s) for s in range(NSTREAMS)],
        out_specs=[
            pl.BlockSpec((ROW_BLK,), lambda i: (i,)),
            pl.BlockSpec((N,), lambda i: (0,)),
        ],
        out_shape=[
            jax.ShapeDtypeStruct((N,), jnp.int32),
            jax.ShapeDtypeStruct((N,), jnp.float32),
        ],
    )(*([adj] * NSTREAMS))


def _lane_allreduce(v, lane_iota):
    for s in (8, 4, 2, 1):
        perm = (lane_iota + s) % LANES
        v = v + jnp.take_along_axis(v, perm, axis=0)
    return v


def _combine_body(cp_ref, ind_ref, out_ref):
    out_ref[...] = (ind_ref[...] + jnp.sum(cp_ref[...], axis=0)).astype(jnp.int32)


def _combine(colpart, ind_tc):
    return pl.pallas_call(
        _combine_body,
        out_shape=jax.ShapeDtypeStruct((N,), jnp.int32),
    )(colpart, ind_tc)


_SC_MESH = plsc.VectorSubcoreMesh(core_axis_name="c", subcore_axis_name="s")


@functools.partial(
    pl.kernel,
    mesh=_SC_MESH,
    out_type=(
        jax.ShapeDtypeStruct((SC_ROWS,), jnp.int32),
        jax.ShapeDtypeStruct((NW, N), jnp.float32),
    ),
    scratch_types=[
        pltpu.VMEM((RCHUNK, N), jnp.float32),
        pltpu.VMEM((RCHUNK, N), jnp.float32),
        pltpu.VMEM((N,), jnp.float32),
        pltpu.VMEM((RPT,), jnp.int32),
        pltpu.SemaphoreType.DMA,
        pltpu.SemaphoreType.DMA,
    ],
)
def _sc_reduce(adj, outd_sc, colpart, buf0, buf1, colacc, rows_out,
               sem0, sem1):
    wid = lax.axis_index("s") * NC + lax.axis_index("c")
    row0 = SPLIT + wid * RPT
    bufs = (buf0, buf1)
    sems = (sem0, sem1)

    zero = jnp.zeros((LANES,), jnp.float32)
    lane_iota = lax.iota(jnp.int32, LANES)

    def zbody(c, carry):
        colacc[pl.ds(c * LANES, LANES)] = zero
        return carry

    lax.fori_loop(0, N // LANES, zbody, 0)

    cps = [pltpu.async_copy(adj.at[pl.ds(row0 + g * RCHUNK, RCHUNK), :],
                            bufs[g % 2], sems[g % 2])
           for g in range(2)]

    n_groups = RPT // RCHUNK
    rv = zero
    for g in range(n_groups):
        buf = bufs[g % 2]
        cps[g % 2].wait()

        def cbody(c, raccs, buf=buf):
            sl = pl.ds(c * LANES, LANES)
            vs = [buf[r, sl] for r in range(RCHUNK)]
            csum = (((vs[0] + vs[1]) + (vs[2] + vs[3]))
                    + ((vs[4] + vs[5]) + (vs[6] + vs[7])))
            plsc.addupdate(colacc.at[sl], csum)
            return tuple(raccs[r] + vs[r] for r in range(RCHUNK))

        raccs = plsc.parallel_loop(
            0, N // LANES, carry=(zero,) * RCHUNK, unroll=2)(cbody)
        for r in range(RCHUNK):
            lane = (g % 2) * RCHUNK + r
            rv = jnp.where(lane_iota == lane,
                           _lane_allreduce(raccs[r], lane_iota), rv)
        if g % 2 == 1:
            rows_out[pl.ds((g // 2) * LANES, LANES)] = rv.astype(jnp.int32)
            rv = zero
        if g + 2 < n_groups:
            cps[g % 2] = pltpu.async_copy(
                adj.at[pl.ds(row0 + (g + 2) * RCHUNK, RCHUNK), :],
                buf, sems[g % 2])

    pltpu.sync_copy(rows_out, outd_sc.at[pl.ds(wid * RPT, RPT)])
    pltpu.sync_copy(colacc, colpart.at[wid])


@functools.partial(
    pl.kernel,
    mesh=_SC_MESH,
    out_type=jax.ShapeDtypeStruct((N, D), jnp.float32),
    scratch_types=[
        pltpu.VMEM((B_PER_W,), jnp.int32),
        pltpu.VMEM((B_PER_W,), jnp.int32),
        pltpu.VMEM((B_PER_W,), jnp.int32),
        pltpu.VMEM((B_PER_W, D), jnp.float32),
        pltpu.VMEM((B_PER_W, D), jnp.float32),
        pltpu.SemaphoreType.DMA,
        pltpu.SemaphoreType.DMA,
    ],
)
def _gather_add(ind_hbm, outd_tc, outd_sc, in_tab, out_tab, out_hbm,
                idx_i, idx_o, idx_o2, rows_i, rows_o, sem_i, sem_o):
    wid = lax.axis_index("s") * NC + lax.axis_index("c")
    base = wid * B_PER_W

    li = pltpu.async_copy(ind_hbm.at[pl.ds(base, B_PER_W)], idx_i, sem_i)
    lt = pltpu.async_copy(outd_tc.at[pl.ds(base, B_PER_W)], idx_o, sem_o)
    off_sc = pl.multiple_of(jnp.maximum(base - SPLIT, 0), 8)
    ls = pltpu.async_copy(outd_sc.at[pl.ds(off_sc, B_PER_W)], idx_o2, sem_o)
    lt.wait()
    ls.wait()

    @pl.when(wid >= SPLIT // B_PER_W)
    def _():
        for c in range(B_PER_W // LANES):
            sl = pl.ds(c * LANES, LANES)
            idx_o[sl] = idx_o2[sl]

    li.wait()

    ci = pltpu.async_copy(in_tab.at[idx_i], rows_i, sem_i)
    co = pltpu.async_copy(out_tab.at[idx_o], rows_o, sem_o)
    ci.wait()
    co.wait()

    def add_body(r, carry):
        for c in range(D // LANES):
            sl = (r, pl.ds(c * LANES, LANES))
            rows_i[sl] = rows_i[sl] + rows_o[sl]
        return carry

    lax.fori_loop(0, B_PER_W, add_body, 0)
    pltpu.sync_copy(rows_i, out_hbm.at[pl.ds(base, B_PER_W)])


def kernel(dense_adj_mx, in_table, out_table):
    outd_tc, ind_tc = _tc_degrees(dense_adj_mx)
    outd_sc, colpart = _sc_reduce(dense_adj_mx)
    indegree = _combine(colpart, ind_tc)
    return _gather_add(indegree, outd_tc, outd_sc, in_table, out_table)


# speedup vs baseline: 1.1307x; 1.1307x over previous
import functools

import jax
import jax.numpy as jnp
from jax import lax
from jax.experimental import pallas as pl
from jax.experimental.pallas import tpu as pltpu
from jax.experimental.pallas import tpu_sc as plsc

N = 4096
D = 128
ROW_BLK = 512
GRID = N // ROW_BLK
HALF = N // 2

NC = 2
NS = 16
NW = NC * NS
B_PER_W = N // NW
HW = B_PER_W // 2
LANES = 16


NSTREAMS = 2
CW = N // NSTREAMS


def _degree_body(*refs):
    adj_refs = refs[:NSTREAMS]
    outd_ref, ind_ref = refs[NSTREAMS], refs[NSTREAMS + 1]
    i = pl.program_id(0)
    blks = [r[...] for r in adj_refs]
    row = jnp.sum(blks[0], axis=1)
    for b in blks[1:]:
        row = row + jnp.sum(b, axis=1)
    outd_ref[...] = row.astype(jnp.int32)
    cols = [jnp.sum(b, axis=0).astype(jnp.int32) for b in blks]

    @pl.when(i == 0)
    def _():
        for s in range(NSTREAMS):
            ind_ref[pl.ds(s * CW, CW)] = cols[s]

    @pl.when(i > 0)
    def _():
        for s in range(NSTREAMS):
            ind_ref[pl.ds(s * CW, CW)] = ind_ref[pl.ds(s * CW, CW)] + cols[s]


def _degrees(adj):
    def make_spec(s):
        return pl.BlockSpec((ROW_BLK, CW), lambda i, s=s: (i, s))

    return pl.pallas_call(
        _degree_body,
        grid=(GRID,),
        in_specs=[make_spec(s) for s in range(NSTREAMS)],
        out_specs=[
            pl.BlockSpec((ROW_BLK,), lambda i: (i,)),
            pl.BlockSpec((N,), lambda i: (0,)),
        ],
        out_shape=[
            jax.ShapeDtypeStruct((N,), jnp.int32),
            jax.ShapeDtypeStruct((N,), jnp.int32),
        ],
    )(*([adj] * NSTREAMS))


_SC_MESH = plsc.VectorSubcoreMesh(core_axis_name="c", subcore_axis_name="s")


@functools.partial(
    pl.kernel,
    mesh=_SC_MESH,
    out_type=jax.ShapeDtypeStruct((N, D), jnp.float32),
    scratch_types=[
        pltpu.VMEM((HW, ), jnp.int32),
        pltpu.VMEM((HW, ), jnp.int32),
        pltpu.VMEM((HW, ), jnp.int32),
        pltpu.VMEM((HW, ), jnp.int32),
        pltpu.VMEM((B_PER_W, D), jnp.float32),
        pltpu.VMEM((B_PER_W, D), jnp.float32),
        pltpu.SemaphoreType.DMA,
        pltpu.SemaphoreType.DMA,
    ],
)
def _gather_add(ind_hbm, outd_hbm, in_tab, out_tab, out_hbm,
                idx_ia, idx_ib, idx_oa, idx_ob, rows_i, rows_o,
                sem_a, sem_b):
    wid = lax.axis_index("s") * NC + lax.axis_index("c")
    base = wid * B_PER_W

    li_a = pltpu.async_copy(ind_hbm.at[pl.ds(base, HW)], idx_ia, sem_a)
    lo_a = pltpu.async_copy(outd_hbm.at[pl.ds(base, HW)], idx_oa, sem_a)
    li_b = pltpu.async_copy(ind_hbm.at[pl.ds(base + HW, HW)], idx_ib, sem_b)
    lo_b = pltpu.async_copy(outd_hbm.at[pl.ds(base + HW, HW)], idx_ob, sem_b)
    li_a.wait()
    lo_a.wait()
    g1 = pltpu.async_copy(in_tab.at[idx_ia], rows_i.at[pl.ds(0, HW)], sem_a)
    g2 = pltpu.async_copy(out_tab.at[idx_oa], rows_o.at[pl.ds(0, HW)], sem_a)
    li_b.wait()
    lo_b.wait()
    g3 = pltpu.async_copy(in_tab.at[idx_ib], rows_i.at[pl.ds(HW, HW)], sem_b)
    g4 = pltpu.async_copy(out_tab.at[idx_ob], rows_o.at[pl.ds(HW, HW)], sem_b)

    def add_body(r, carry):
        for c in range(D // LANES):
            sl = (r, pl.ds(c * LANES, LANES))
            rows_i[sl] = rows_i[sl] + rows_o[sl]
        return carry

    g1.wait()
    g2.wait()
    lax.fori_loop(0, HW, add_body, 0)
    wb_a = pltpu.async_copy(rows_i.at[pl.ds(0, HW)],
                            out_hbm.at[pl.ds(base, HW)], sem_a)
    g3.wait()
    g4.wait()
    lax.fori_loop(HW, B_PER_W, add_body, 0)
    pltpu.sync_copy(rows_i.at[pl.ds(HW, HW)], out_hbm.at[pl.ds(base + HW, HW)])
    wb_a.wait()


def kernel(dense_adj_mx, in_table, out_table):
    outdegree, indegree = _degrees(dense_adj_mx)
    return _gather_add(indegree, outdegree, in_table, out_table)


# speedup vs baseline: 1.1407x; 1.0089x over previous
import functools

import jax
import jax.numpy as jnp
from jax import lax
from jax.experimental import pallas as pl
from jax.experimental.pallas import tpu as pltpu
from jax.experimental.pallas import tpu_sc as plsc

N = 4096
D = 128
ROW_BLK = 512
GRID = N // ROW_BLK
HALF = N // 2

NC = 2
NS = 16
NW = NC * NS
B_PER_W = N // NW
LANES = 16


NSTREAMS = 2
CW = N // NSTREAMS


def _degree_body(*refs):
    adj_refs = refs[:NSTREAMS]
    outd_ref, ind_ref = refs[NSTREAMS], refs[NSTREAMS + 1]
    i = pl.program_id(0)
    blks = [r[...] for r in adj_refs]
    row = jnp.sum(blks[0], axis=1)
    for b in blks[1:]:
        row = row + jnp.sum(b, axis=1)
    outd_ref[...] = row.astype(jnp.int32)
    cols = [jnp.sum(b, axis=0).astype(jnp.int32) for b in blks]

    @pl.when(i == 0)
    def _():
        for s in range(NSTREAMS):
            ind_ref[pl.ds(s * CW, CW)] = cols[s]

    @pl.when(i > 0)
    def _():
        for s in range(NSTREAMS):
            ind_ref[pl.ds(s * CW, CW)] = ind_ref[pl.ds(s * CW, CW)] + cols[s]


def _degrees(adj):
    def make_spec(s):
        return pl.BlockSpec((ROW_BLK, CW), lambda i, s=s: (i, s))

    return pl.pallas_call(
        _degree_body,
        grid=(GRID,),
        in_specs=[make_spec(s) for s in range(NSTREAMS)],
        out_specs=[
            pl.BlockSpec((ROW_BLK,), lambda i: (i,)),
            pl.BlockSpec((N,), lambda i: (0,)),
        ],
        out_shape=[
            jax.ShapeDtypeStruct((N,), jnp.int32),
            jax.ShapeDtypeStruct((N,), jnp.int32),
        ],
    )(*([adj] * NSTREAMS))


_SC_MESH = plsc.VectorSubcoreMesh(core_axis_name="c", subcore_axis_name="s")


@functools.partial(
    pl.kernel,
    mesh=_SC_MESH,
    out_type=jax.ShapeDtypeStruct((N, D), jnp.float32),
    scratch_types=[
        pltpu.VMEM((B_PER_W,), jnp.int32),
        pltpu.VMEM((B_PER_W,), jnp.int32),
        pltpu.VMEM((B_PER_W, D), jnp.float32),
        pltpu.VMEM((B_PER_W, D), jnp.float32),
        pltpu.SemaphoreType.DMA,
        pltpu.SemaphoreType.DMA,
    ],
)
def _gather_add(ind_hbm, outd_hbm, in_tab, out_tab, out_hbm,
                idx_i, idx_o, rows_i, rows_o, sem_i, sem_o):
    wid = lax.axis_index("s") * NC + lax.axis_index("c")
    base = wid * B_PER_W
    li = pltpu.async_copy(ind_hbm.at[pl.ds(base, B_PER_W)], idx_i, sem_i)
    lo = pltpu.async_copy(outd_hbm.at[pl.ds(base, B_PER_W)], idx_o, sem_o)
    li.wait()
    lo.wait()
    ci = pltpu.async_copy(in_tab.at[idx_i], rows_i, sem_i)
    co = pltpu.async_copy(out_tab.at[idx_o], rows_o, sem_o)
    ci.wait()
    co.wait()

    def body(r, carry):
        for c in range(D // LANES):
            sl = (r, pl.ds(c * LANES, LANES))
            rows_i[sl] = rows_i[sl] + rows_o[sl]
        return carry

    lax.fori_loop(0, B_PER_W, body, 0)
    pltpu.sync_copy(rows_i, out_hbm.at[pl.ds(base, B_PER_W)])


def kernel(dense_adj_mx, in_table, out_table):
    outdegree, indegree = _degrees(dense_adj_mx)
    return _gather_add(indegree, outdegree, in_table, out_table)


# speedup vs baseline: 1.1556x; 1.0131x over previous
import functools

import jax
import jax.numpy as jnp
from jax import lax
from jax.experimental import pallas as pl
from jax.experimental.pallas import tpu as pltpu
from jax.experimental.pallas import tpu_sc as plsc

N = 4096
D = 128
ROW_BLK = 512
GRID = N // ROW_BLK
HALF = N // 2

NC = 2
NS = 16
NW = NC * NS
B_PER_W = N // NW
LANES = 16


NSTREAMS = 2
CW = N // NSTREAMS


def _degree_body(*refs):
    adj_refs = refs[:NSTREAMS]
    outd_ref, ind_ref = refs[NSTREAMS], refs[NSTREAMS + 1]
    i = pl.program_id(0)
    blks = [r[...] for r in adj_refs]
    row = jnp.sum(blks[0], axis=1)
    for b in blks[1:]:
        row = row + jnp.sum(b, axis=1)
    outd_ref[...] = row.astype(jnp.int32)
    cols = [jnp.sum(b, axis=0).astype(jnp.int32) for b in blks]

    @pl.when(i == 0)
    def _():
        for s in range(NSTREAMS):
            ind_ref[pl.ds(s * CW, CW)] = cols[s]

    @pl.when(i > 0)
    def _():
        for s in range(NSTREAMS):
            ind_ref[pl.ds(s * CW, CW)] = ind_ref[pl.ds(s * CW, CW)] + cols[s]


def _degrees(adj):
    def make_spec(s):
        return pl.BlockSpec((ROW_BLK, CW), lambda i, s=s: (i, s))

    return pl.pallas_call(
        _degree_body,
        grid=(GRID,),
        in_specs=[make_spec(s) for s in range(NSTREAMS)],
        out_specs=[
            pl.BlockSpec((ROW_BLK,), lambda i: (i,)),
            pl.BlockSpec((N,), lambda i: (0,)),
        ],
        out_shape=[
            jax.ShapeDtypeStruct((N,), jnp.int32),
            jax.ShapeDtypeStruct((N,), jnp.int32),
        ],
    )(*([adj] * NSTREAMS))


_SC_MESH = plsc.VectorSubcoreMesh(core_axis_name="c", subcore_axis_name="s")


@functools.partial(
    pl.kernel,
    mesh=_SC_MESH,
    out_type=jax.ShapeDtypeStruct((N, D), jnp.float32),
    scratch_types=[
        pltpu.VMEM((B_PER_W,), jnp.int32),
        pltpu.VMEM((B_PER_W,), jnp.int32),
        pltpu.VMEM((B_PER_W, D), jnp.float32),
        pltpu.VMEM((B_PER_W, D), jnp.float32),
        pltpu.SemaphoreType.DMA,
        pltpu.SemaphoreType.DMA,
    ],
)
def _gather_add(ind_hbm, outd_hbm, in_tab, out_tab, out_hbm,
                idx_i, idx_o, rows_i, rows_o, sem_i, sem_o):
    wid = lax.axis_index("s") * NC + lax.axis_index("c")
    base = wid * B_PER_W
    li = pltpu.async_copy(ind_hbm.at[pl.ds(base, B_PER_W)], idx_i, sem_i)
    lo = pltpu.async_copy(outd_hbm.at[pl.ds(base, B_PER_W)], idx_o, sem_o)
    li.wait()
    lo.wait()
    ci = pltpu.async_copy(in_tab.at[idx_i], rows_i, sem_i)
    co = pltpu.async_copy(out_tab.at[idx_o], rows_o, sem_o)
    ci.wait()
    co.wait()

    @plsc.parallel_loop(0, B_PER_W, unroll=2)
    def _(r):
        for c in range(D // LANES):
            sl = (r, pl.ds(c * LANES, LANES))
            rows_i[sl] = rows_i[sl] + rows_o[sl]
    pltpu.sync_copy(rows_i, out_hbm.at[pl.ds(base, B_PER_W)])


def kernel(dense_adj_mx, in_table, out_table):
    outdegree, indegree = _degrees(dense_adj_mx)
    return _gather_add(indegree, outdegree, in_table, out_table)
